# baseline (speedup 1.0000x reference)
import functools
import math

import jax
import jax.numpy as jnp
from jax import lax
from jax.experimental import pallas as pl
from jax.experimental.pallas import tpu as pltpu
from jax.experimental.pallas import tpu_sc as plsc

N = 10000
E = 320000
D = 128
H = 128

C = 80
B = 8
NW_STATIC = 32
NCHUNK = -(-E // (C * B * NW_STATIC)) * B * NW_STATIC
E_PAD = NCHUNK * C - E
NCH_T = NCHUNK // NW_STATIC
NBLK = NCH_T // B


def _sc_aggregate(x, dst3d, src3d, val3d):
    info = plsc.get_sparse_core_info()
    NC, NS = info.num_cores, info.num_subcores
    rows_per_tile = (N // NS) // 8 * 8
    rem_rows = N - NS * rows_per_tile
    rem_base = NS * rows_per_tile

    mesh = plsc.VectorSubcoreMesh(core_axis_name="c", subcore_axis_name="s")

    @functools.partial(
        pl.kernel,
        mesh=mesh,
        out_type=(
            jax.ShapeDtypeStruct((N, D), jnp.float32),
            jax.ShapeDtypeStruct((N, D), jnp.float32),
        ),
        scratch_types=[
            pltpu.VMEM((2 * B, 1, C), jnp.int32),
            pltpu.VMEM((2 * B, 1, C), jnp.int32),
            pltpu.VMEM((2 * B, 1, C), jnp.float32),
            pltpu.VMEM((C, D), jnp.float32),
            pltpu.VMEM((C, D), jnp.float32),
            pltpu.VMEM((C, D), jnp.float32),
            pltpu.VMEM((C, D), jnp.float32),
            pltpu.VMEM_SHARED((N, D), jnp.float32),
            pltpu.SemaphoreType.DMA,
            pltpu.SemaphoreType.DMA,
            pltpu.SemaphoreType.DMA,
            pltpu.SemaphoreType.DMA,
            pltpu.SemaphoreType.DMA,
        ],
    )
    def agg_kernel(x_hbm, dst_hbm, src_hbm, val_hbm, out0, out1,
                   db, sb, vb, rg0, rg1, rs0, rs1,
                   acc_sh, gsem0, gsem1, ssem0, ssem1, isem):
        cid = lax.axis_index("c")
        sid = lax.axis_index("s")
        wid = sid * NC + cid
        rg = (rg0, rg1)
        rs = (rs0, rs1)
        gsem = (gsem0, gsem1)
        ssem = (ssem0, ssem1)
        gbase = wid * NCH_T

        def start_idx(blk):
            sl = pl.ds(gbase + blk * B, B)
            dsl = pl.ds((blk % 2) * B, B)
            pltpu.async_copy(dst_hbm.at[sl], db.at[dsl], isem)
            pltpu.async_copy(src_hbm.at[sl], sb.at[dsl], isem)
            pltpu.async_copy(val_hbm.at[sl], vb.at[dsl], isem)

        def wait_idx():
            sl = pl.ds(0, B)
            dsl = pl.ds(0, B)
            pltpu.make_async_copy(dst_hbm.at[sl], db.at[dsl], isem).wait()
            pltpu.make_async_copy(src_hbm.at[sl], sb.at[dsl], isem).wait()
            pltpu.make_async_copy(val_hbm.at[sl], vb.at[dsl], isem).wait()

        def rowq(i):
            return ((i // B) % 2) * B + (i % B)

        start_idx(0)

        def zrow(r, _):
            for k8 in range(D // 16):
                rg0[r, pl.ds(16 * k8, 16)] = jnp.zeros((16,), jnp.float32)
            return 0
        lax.fori_loop(0, C, zrow, 0)
        base = sid * rows_per_tile
        ztail = rows_per_tile - 7 * C
        for i in range(7):
            pltpu.sync_copy(rg0, acc_sh.at[pl.ds(base + i * C, C)])
        pltpu.sync_copy(rg0.at[pl.ds(0, ztail)],
                        acc_sh.at[pl.ds(base + 7 * C, ztail)])

        @pl.when(sid == 0)
        def _():
            pltpu.sync_copy(rg0.at[pl.ds(0, rem_rows)],
                            acc_sh.at[pl.ds(rem_base, rem_rows)])

        wait_idx()
        plsc.subcore_barrier()

        pltpu.async_copy(x_hbm.at[sb.at[0, 0]], rg0, gsem0)
        pltpu.async_copy(x_hbm.at[sb.at[1, 0]], rg1, gsem1)

        def pair_body(t, _):
            i0 = 2 * t
            q = i0 // B
            c0 = i0 % B

            @pl.when((c0 == 2) & (q < NBLK - 1))
            def _():
                start_idx(q + 1)

            @pl.when((c0 == B - 2) & (q < NBLK - 1))
            def _():
                wait_idx()

            for b in range(2):
                i = i0 + b
                r = rowq(i)
                pltpu.make_async_copy(
                    x_hbm.at[sb.at[r, 0]], rg[b], gsem[b]).wait()

                @pl.when(i >= 2)
                def _():
                    pltpu.make_async_copy(
                        rs[b], acc_sh.at[db.at[r, 0]], ssem[b]).wait()

                def scale(gg, _):
                    val16 = vb[r, 0, pl.ds(gg * 16, 16)]
                    for l in range(16):
                        v = val16[l]
                        j = gg * 16 + l
                        for k8 in range(D // 16):
                            sl = pl.ds(16 * k8, 16)
                            rs[b][j, sl] = rg[b][j, sl] * v
                    return 0
                lax.fori_loop(0, C // 16, scale, 0)

                @pl.when(i + 2 < NCH_T)
                def _():
                    pltpu.async_copy(x_hbm.at[sb.at[rowq(i + 2), 0]],
                                     rg[b], gsem[b])
                pltpu.async_copy(rs[b], acc_sh.at[db.at[r, 0]],
                                 ssem[b], add=True)
            return 0
        lax.fori_loop(0, NCH_T // 2, pair_body, 0)
        for b in range(2):
            r = rowq(NCH_T - 2 + b)
            pltpu.make_async_copy(
                rs[b], acc_sh.at[db.at[r, 0]], ssem[b]).wait()
        plsc.subcore_barrier()

        def copy_out(out_ref):
            for i in range(7):
                pltpu.sync_copy(acc_sh.at[pl.ds(base + i * C, C)],
                                out_ref.at[pl.ds(base + i * C, C)])
            pltpu.sync_copy(acc_sh.at[pl.ds(base + 7 * C, ztail)],
                            out_ref.at[pl.ds(base + 7 * C, ztail)])

            @pl.when(sid == 0)
            def _():
                pltpu.sync_copy(acc_sh.at[pl.ds(rem_base, rem_rows)],
                                out_ref.at[pl.ds(rem_base, rem_rows)])

        @pl.when(cid == 0)
        def _():
            copy_out(out0)

        @pl.when(cid == 1)
        def _():
            copy_out(out1)

    return agg_kernel(x, dst3d, src3d, val3d)


_BR = 1000
_INV_SQRT2 = 1.0 / math.sqrt(2.0)


def _ln_gelu(h, g, be):
    mu = jnp.mean(h, axis=-1, keepdims=True)
    var = jnp.mean((h - mu) ** 2, axis=-1, keepdims=True)
    h = (h - mu) / jnp.sqrt(var + 1e-5) * g + be
    return 0.5 * h * (1.0 + lax.erf(h * _INV_SQRT2))


def _dense1_body(x_ref, a0_ref, a1_ref, W_ref, b_ref, g_ref, be_ref, o_ref):
    h = x_ref[...] + a0_ref[...] + a1_ref[...]
    h = jnp.dot(h, W_ref[...], preferred_element_type=jnp.float32) + b_ref[...]
    o_ref[...] = _ln_gelu(h, g_ref[...], be_ref[...])


def _dense2_body(x_ref, a0_ref, a1_ref, W2_ref, b2_ref, g2_ref, be2_ref,
                 Wf_ref, bf_ref, o_ref):
    h = x_ref[...] + a0_ref[...] + a1_ref[...]
    h = jnp.dot(h, W2_ref[...], preferred_element_type=jnp.float32) + b2_ref[...]
    h = _ln_gelu(h, g2_ref[...], be2_ref[...])
    o_ref[...] = jnp.dot(h, Wf_ref[...], preferred_element_type=jnp.float32) + bf_ref[...]


def _row_spec():
    return pl.BlockSpec((_BR, D), lambda i: (i, 0))


def _rep_spec(shape):
    return pl.BlockSpec(shape, lambda i: (0,) * len(shape))


def _dense1(x, a0, a1, W, b, g, be):
    return pl.pallas_call(
        _dense1_body,
        grid=(N // _BR,),
        in_specs=[_row_spec(), _row_spec(), _row_spec(),
                  _rep_spec((D, H)), _rep_spec((1, H)), _rep_spec((1, H)),
                  _rep_spec((1, H))],
        out_specs=_row_spec(),
        out_shape=jax.ShapeDtypeStruct((N, H), jnp.float32),
    )(x, a0, a1, W, b.reshape(1, H), g.reshape(1, H), be.reshape(1, H))


def _dense2(x, a0, a1, W2, b2, g2, be2, Wf, bf):
    return pl.pallas_call(
        _dense2_body,
        grid=(N // _BR,),
        in_specs=[_row_spec(), _row_spec(), _row_spec(),
                  _rep_spec((H, H)), _rep_spec((1, H)), _rep_spec((1, H)),
                  _rep_spec((1, H)),
                  _rep_spec((H, D)), _rep_spec((1, D))],
        out_specs=_row_spec(),
        out_shape=jax.ShapeDtypeStruct((N, D), jnp.float32),
    )(x, a0, a1, W2, b2.reshape(1, H), g2.reshape(1, H), be2.reshape(1, H),
      Wf, bf.reshape(1, D))


def kernel(node_features, adj_indices, adj_values, W1, b1, g1, be1,
           W2, b2, g2, be2, Wf, bf):
    zpad_i = jnp.zeros((E_PAD,), jnp.int32)
    dst3d = jnp.concatenate(
        [adj_indices[0].astype(jnp.int32), zpad_i]).reshape(NCHUNK, 1, C)
    src3d = jnp.concatenate(
        [adj_indices[1].astype(jnp.int32), zpad_i]).reshape(NCHUNK, 1, C)
    val3d = jnp.concatenate(
        [adj_values, jnp.zeros((E_PAD,), jnp.float32)]).reshape(NCHUNK, 1, C)

    a0, a1 = _sc_aggregate(node_features, dst3d, src3d, val3d)
    x1 = _dense1(node_features, a0, a1, W1, b1, g1, be1)
    c0, c1 = _sc_aggregate(x1, dst3d, src3d, val3d)
    return _dense2(x1, c0, c1, W2, b2, g2, be2, Wf, bf)
